# baseline (speedup 1.0000x reference)
import functools
import math

import jax
import jax.numpy as jnp
from jax import lax
from jax.experimental import pallas as pl
from jax.experimental.pallas import tpu as pltpu
from jax.experimental.pallas import tpu_sc as plsc

_D = 64
_SCALE = math.sqrt(_D)
_L = 16


def _iota16(g):
    return jnp.arange(g * _L, (g + 1) * _L, dtype=jnp.int32)


@functools.lru_cache(maxsize=None)
def _build_transpose(v_rows: int):
    info = plsc.get_sparse_core_info()
    nw = info.num_cores * info.num_subcores
    chunk = 256
    n_full = v_rows // chunk
    n_even = (n_full // nw) * nw
    slots = n_even // nw
    rem_full = n_full - n_even
    tail = v_rows - n_full * chunk
    mesh = plsc.VectorSubcoreMesh(core_axis_name="c", subcore_axis_name="s")

    @functools.partial(
        pl.kernel,
        out_type=jax.ShapeDtypeStruct((v_rows // 2, 128), jnp.float32),
        mesh=mesh,
        scratch_types=[
            pltpu.VMEM((_D, chunk), jnp.float32),
            pltpu.VMEM((_D, chunk), jnp.float32),
            pltpu.VMEM((chunk // 2, 128), jnp.float32),
            pltpu.VMEM((chunk // 2, 128), jnp.float32),
            pltpu.SemaphoreType.DMA,
            pltpu.SemaphoreType.DMA,
            pltpu.SemaphoreType.DMA,
            pltpu.SemaphoreType.DMA,
        ],
        compiler_params=pltpu.CompilerParams(needs_layout_passes=False),
    )
    def tr_kernel(src_hbm, tail_hbm, out_hbm, src0, src1, dst0, dst1,
                  gs0, gs1, ws0, ws1):
        wid = lax.axis_index("s") * info.num_cores + lax.axis_index("c")
        srcs = (src0, src1)
        dsts = (dst0, dst1)
        gsems = (gs0, gs1)
        wsems = (ws0, ws1)
        n_groups = chunk // _L

        tvecs = [_iota16(g) for g in range(n_groups)]
        rowbases = [lax.shift_right_logical(t, 1) for t in tvecs]
        colbases = [lax.shift_left(lax.bitwise_and(t, 1), 6) for t in tvecs]

        def chunk_id(slot):
            return wid + nw * slot

        def start_read(slot, b):
            v0 = pl.multiple_of(chunk_id(slot) * chunk, 128)
            pltpu.async_copy(src_hbm.at[:, pl.ds(v0, chunk)], srcs[b],
                             gsems[b])

        def wait_read(b):
            pltpu.make_async_copy(src_hbm.at[:, pl.ds(0, chunk)], srcs[b],
                                  gsems[b]).wait()

        def wait_write(b):
            pltpu.make_async_copy(dsts[b],
                                  out_hbm.at[pl.ds(0, chunk // 2)],
                                  wsems[b]).wait()

        def transpose_chunk(b, groups=n_groups):
            sv = srcs[b]
            dv = dsts[b]
            for gb in range(0, groups, 8):
                hi = min(gb + 8, groups)
                rb = rowbases[gb:hi]
                cb = colbases[gb:hi]
                tv = tvecs[gb:hi]

                @plsc.parallel_loop(0, _D, unroll=2)
                def d_loop(d):
                    drow = jnp.zeros((_L,), jnp.int32) + d
                    for g in range(len(tv)):
                        vals = plsc.load_gather(sv, [drow, tv[g]])
                        plsc.store_scatter(dv, [rb[g], cb[g] + d], vals)

        def start_write(slot, b):
            j0 = pl.multiple_of(chunk_id(slot) * (chunk // 2), 8)
            pltpu.async_copy(dsts[b], out_hbm.at[pl.ds(j0, chunk // 2)],
                             wsems[b])

        start_read(0, 0)
        start_read(1, 1)
        wait_read(0)
        transpose_chunk(0)
        start_write(0, 0)
        start_read(2, 0)
        wait_read(1)
        transpose_chunk(1)
        start_write(1, 1)
        start_read(3, 1)

        def pair_body(i, carry):
            g = 2 * i
            for b in range(2):
                wait_read(b)
                wait_write(b)
                transpose_chunk(b)
                start_write(g + b, b)
                start_read(g + b + 2, b)
            return carry

        lax.fori_loop(1, slots // 2 - 1, pair_body, 0)
        g = slots - 2
        for b in range(2):
            wait_read(b)
            wait_write(b)
            transpose_chunk(b)
            start_write(g + b, b)
        wait_write(0)
        wait_write(1)

        @pl.when(wid == 0)
        def _tail():
            for k in range(rem_full):
                ci = n_even + k
                v0 = ci * chunk
                pltpu.sync_copy(src_hbm.at[:, pl.ds(v0, chunk)], src0)
                transpose_chunk(0)
                pltpu.sync_copy(dst0,
                                out_hbm.at[pl.ds(ci * (chunk // 2),
                                                 chunk // 2)])
            if tail:
                pltpu.sync_copy(tail_hbm, dst0.at[pl.ds(0, tail // 2)])
                pltpu.sync_copy(dst0.at[pl.ds(0, tail // 2)],
                                out_hbm.at[pl.ds(n_full * chunk // 2,
                                                 tail // 2)])

    return tr_kernel


@functools.lru_cache(maxsize=None)
def _build_gather(n_b: int, n_s: int):
    info = plsc.get_sparse_core_info()
    nw = info.num_cores * info.num_subcores
    bc = 128
    assert n_s % 8 == 0 and n_b % bc == 0
    n_bchunks = n_b // bc
    n_units = (n_s // 8) * n_bchunks
    assert n_units % nw == 0
    units_per_w = n_units // nw
    n_groups = bc // _L
    mesh = plsc.VectorSubcoreMesh(core_axis_name="c", subcore_axis_name="s")

    @functools.partial(
        pl.kernel,
        out_type=jax.ShapeDtypeStruct((n_s, _D, n_b), jnp.float32),
        mesh=mesh,
        scratch_types=[
            pltpu.VMEM((8, bc), jnp.int32),
            pltpu.VMEM((bc,), jnp.int32),
            pltpu.VMEM((bc,), jnp.int32),
            pltpu.VMEM((bc, 128), jnp.float32),
            pltpu.VMEM((bc, 128), jnp.float32),
            pltpu.VMEM((_D, bc), jnp.float32),
            pltpu.VMEM((_D, bc), jnp.float32),
            pltpu.SemaphoreType.DMA,
            pltpu.SemaphoreType.DMA,
            pltpu.SemaphoreType.DMA,
            pltpu.SemaphoreType.DMA,
        ],
        compiler_params=pltpu.CompilerParams(needs_layout_passes=False),
    )
    def g_kernel(xt_hbm, tbl_hbm, out_hbm, idx_v, j0_v, j1_v,
                 rows0, rows1, blk0, blk1, gs0, gs1, ws0, ws1):
        wid = lax.axis_index("s") * info.num_cores + lax.axis_index("c")
        jbufs = (j0_v, j1_v)
        rowss = (rows0, rows1)
        blks = (blk0, blk1)
        gsems = (gs0, gs1)
        wsems = (ws0, ws1)

        rowb = [_iota16(g) for g in range(n_groups)]

        def load_idx(u):
            oct_ = pl.multiple_of((u // n_bchunks) * 8, 8)
            b0 = pl.multiple_of((u % n_bchunks) * bc, bc)
            pltpu.sync_copy(xt_hbm.at[pl.ds(oct_, 8), pl.ds(b0, bc)],
                            idx_v)

        def start_gather(r, b):
            jb = jbufs[b]
            for g in range(n_groups):
                s = pl.ds(g * _L, _L)
                jb[s] = lax.shift_right_logical(idx_v[r, s], 1)
            pltpu.async_copy(tbl_hbm.at[jb], rowss[b], gsems[b])

        def finish(u, r, b, wait_wb):
            oct_ = u // n_bchunks
            b0 = pl.multiple_of((u % n_bchunks) * bc, bc)
            pltpu.make_async_copy(tbl_hbm.at[jbufs[b]], rowss[b],
                                  gsems[b]).wait()
            rv = rowss[b]
            bv = blks[b]
            p64 = []
            for g in range(n_groups):
                s = pl.ds(g * _L, _L)
                p64.append(lax.shift_left(
                    lax.bitwise_and(idx_v[r, s], 1), 6))
            if wait_wb:
                pltpu.make_async_copy(bv, out_hbm.at[0, :, pl.ds(0, bc)],
                                      wsems[b]).wait()
            for gb in range(0, n_groups, 8):
                pg = p64[gb:gb + 8]
                rg = rowb[gb:gb + 8]

                @plsc.parallel_loop(0, _D, unroll=2)
                def d_loop(d):
                    drow = jnp.zeros((_L,), jnp.int32) + d
                    for g in range(8):
                        vals = plsc.load_gather(rv, [rg[g], pg[g] + d])
                        plsc.store_scatter(bv, [drow, rg[g]],
                                           vals * _SCALE)
            pltpu.async_copy(bv, out_hbm.at[oct_ * 8 + r, :,
                                            pl.ds(b0, bc)],
                             wsems[b])

        def run_unit(u, first):
            load_idx(u)
            start_gather(0, 0)
            for r in range(8):
                b = r % 2
                if r + 1 < 8:
                    start_gather(r + 1, 1 - b)
                finish(u, r, b, wait_wb=(not first) or r >= 2)

        run_unit(wid * units_per_w, True)

        def unit_body(i, carry):
            u = wid * units_per_w + i
            load_idx(u)
            start_gather(0, 0)
            for r in range(8):
                b = r % 2
                if r + 1 < 8:
                    start_gather(r + 1, 1 - b)
                finish(u, r, b, True)
            return carry

        lax.fori_loop(1, units_per_w, unit_body, 0)
        pltpu.make_async_copy(blk0, out_hbm.at[0, :, pl.ds(0, bc)],
                              ws0).wait()
        pltpu.make_async_copy(blk1, out_hbm.at[0, :, pl.ds(0, bc)],
                              ws1).wait()

    return g_kernel


def kernel(x, table):
    n_b, n_s = x.shape
    v_rows = table.shape[0]
    xt = jnp.transpose(x)
    tt = jnp.transpose(table)
    t0 = (v_rows // 256) * 256
    tail2 = table[t0:].reshape((v_rows - t0) // 2, 128)
    packed = _build_transpose(v_rows)(tt, tail2)
    out = _build_gather(n_b, n_s)(xt, packed)
    return jnp.transpose(out, (2, 0, 1))
